# Initial kernel scaffold; baseline (speedup 1.0000x reference)
import functools

import jax
import jax.numpy as jnp
from jax import lax
from jax.experimental import pallas as pl
from jax.experimental.pallas import tpu as pltpu
from jax.experimental.pallas import tpu_sc as plsc

D_MODEL = 64
NUM_CORES = 2
NUM_SUBCORES = 16
NUM_WORKERS = NUM_CORES * NUM_SUBCORES

CHUNK = 128


def _make_gather(total_rows: int):
    b_per_w = total_rows // NUM_WORKERS
    n_chunks = b_per_w // CHUNK
    mesh = plsc.VectorSubcoreMesh(
        core_axis_name="c", subcore_axis_name="s",
        num_cores=NUM_CORES, num_subcores=NUM_SUBCORES,
    )

    @functools.partial(
        pl.kernel,
        mesh=mesh,
        out_type=jax.ShapeDtypeStruct((total_rows, D_MODEL), jnp.float32),
        scratch_types=[
            pltpu.VMEM((CHUNK,), jnp.int32),
            pltpu.VMEM((CHUNK, D_MODEL), jnp.float32),
            pltpu.SemaphoreType.DMA,
        ],
    )
    def gather(idx_hbm, table_hbm, out_hbm, idx_v, rows_v, sem):
        wid = lax.axis_index("s") * NUM_CORES + lax.axis_index("c")
        base = wid * b_per_w

        @pl.loop(0, n_chunks)
        def _(i):
            off = base + i * CHUNK
            pltpu.sync_copy(idx_hbm.at[pl.ds(off, CHUNK)], idx_v)
            pltpu.async_copy(table_hbm.at[idx_v], rows_v, sem).wait()
            pltpu.sync_copy(rows_v, out_hbm.at[pl.ds(off, CHUNK)])

    return gather


def kernel(x, weight):
    idx = x.reshape(-1).astype(jnp.int32)
    out = _make_gather(idx.shape[0])(idx, weight)
    return out.reshape(x.shape + (D_MODEL,))


# speedup vs baseline: 1.5720x; 1.5720x over previous
import functools

import jax
import jax.numpy as jnp
from jax import lax
from jax.experimental import pallas as pl
from jax.experimental.pallas import tpu as pltpu
from jax.experimental.pallas import tpu_sc as plsc

D_MODEL = 64
NUM_CORES = 2
NUM_SUBCORES = 16
NUM_WORKERS = NUM_CORES * NUM_SUBCORES

CHUNK = 128


def _make_gather(total_rows: int):
    b_per_w = total_rows // NUM_WORKERS
    n_chunks = b_per_w // CHUNK
    mesh = plsc.VectorSubcoreMesh(
        core_axis_name="c", subcore_axis_name="s",
        num_cores=NUM_CORES, num_subcores=NUM_SUBCORES,
    )

    @functools.partial(
        pl.kernel,
        mesh=mesh,
        out_type=jax.ShapeDtypeStruct((total_rows, D_MODEL), jnp.float32),
        scratch_types=[
            pltpu.VMEM((CHUNK,), jnp.int32),
            pltpu.VMEM((CHUNK, D_MODEL), jnp.float32),
            pltpu.SemaphoreType.DMA,
        ],
        compiler_params=pltpu.CompilerParams(use_tc_tiling_on_sc=False),
    )
    def gather(idx_hbm, table_hbm, out_hbm, idx_v, rows_v, sem):
        wid = lax.axis_index("s") * NUM_CORES + lax.axis_index("c")
        base = wid * b_per_w

        @pl.loop(0, n_chunks)
        def _(i):
            off = base + i * CHUNK
            pltpu.sync_copy(idx_hbm.at[pl.ds(off, CHUNK)], idx_v)
            pltpu.async_copy(table_hbm.at[idx_v], rows_v, sem).wait()
            pltpu.sync_copy(rows_v, out_hbm.at[pl.ds(off, CHUNK)])

    return gather


def kernel(x, weight):
    idx = x.reshape(-1).astype(jnp.int32)
    out = _make_gather(idx.shape[0])(idx, weight)
    return out.reshape(x.shape + (D_MODEL,))


# speedup vs baseline: 1.8637x; 1.1855x over previous
import functools

import jax
import jax.numpy as jnp
from jax import lax
from jax.experimental import pallas as pl
from jax.experimental.pallas import tpu as pltpu
from jax.experimental.pallas import tpu_sc as plsc

D_MODEL = 64
NUM_CORES = 2
NUM_SUBCORES = 16
NUM_WORKERS = NUM_CORES * NUM_SUBCORES

CHUNK = 128
NBUF = 4
INFLIGHT = 2


def _make_gather(total_rows: int):
    b_per_w = total_rows // NUM_WORKERS
    n_chunks = b_per_w // CHUNK
    mesh = plsc.VectorSubcoreMesh(
        core_axis_name="c", subcore_axis_name="s",
        num_cores=NUM_CORES, num_subcores=NUM_SUBCORES,
    )

    @functools.partial(
        pl.kernel,
        mesh=mesh,
        out_type=jax.ShapeDtypeStruct((total_rows, D_MODEL), jnp.float32),
        scratch_types=[
            pltpu.VMEM((b_per_w,), jnp.int32),
            pltpu.VMEM((NBUF, CHUNK, D_MODEL), jnp.float32),
            pltpu.SemaphoreType.DMA((NBUF,)),
            pltpu.SemaphoreType.DMA((NBUF,)),
        ],
        compiler_params=pltpu.CompilerParams(use_tc_tiling_on_sc=False),
    )
    def gather(idx_hbm, table_hbm, out_hbm, idx_v, rows_v, g_sem, st_sem):
        wid = lax.axis_index("s") * NUM_CORES + lax.axis_index("c")
        base = wid * b_per_w

        pltpu.sync_copy(idx_hbm.at[pl.ds(base, b_per_w)], idx_v)

        def start_gather(chunk, buf):
            pltpu.async_copy(
                table_hbm.at[idx_v.at[pl.ds(chunk * CHUNK, CHUNK)]],
                rows_v.at[buf], g_sem.at[buf])

        def start_store(chunk, buf):
            pltpu.async_copy(
                rows_v.at[buf], out_hbm.at[pl.ds(base + chunk * CHUNK, CHUNK)],
                st_sem.at[buf])

        def wait(sem, buf):
            pltpu.make_async_copy(
                out_hbm.at[pl.ds(base, CHUNK)], rows_v.at[buf], sem.at[buf]
            ).wait()

        for b in range(INFLIGHT):
            start_gather(b, b)

        @pl.loop(0, n_chunks, step=NBUF)
        def _(o):
            for b in range(NBUF):
                i = o + b
                wait(g_sem, b)
                start_store(i, b)
                j = i + INFLIGHT
                bj = (b + INFLIGHT) % NBUF

                @pl.when(j < n_chunks)
                def _():
                    @pl.when(j >= NBUF)
                    def _():
                        wait(st_sem, bj)
                    start_gather(j, bj)

        for b in range(NBUF):
            wait(st_sem, (n_chunks - NBUF + b) % NBUF)

    return gather


def kernel(x, weight):
    idx = x.reshape(-1).astype(jnp.int32)
    out = _make_gather(idx.shape[0])(idx, weight)
    return out.reshape(x.shape + (D_MODEL,))


# speedup vs baseline: 1.8874x; 1.0127x over previous
import functools

import jax
import jax.numpy as jnp
from jax import lax
from jax.experimental import pallas as pl
from jax.experimental.pallas import tpu as pltpu
from jax.experimental.pallas import tpu_sc as plsc

D_MODEL = 64
NUM_CORES = 2
NUM_SUBCORES = 16
NUM_WORKERS = NUM_CORES * NUM_SUBCORES

CHUNK = 128
NBUF = 8
INFLIGHT = 6


def _make_gather(total_rows: int):
    b_per_w = total_rows // NUM_WORKERS
    n_chunks = b_per_w // CHUNK
    mesh = plsc.VectorSubcoreMesh(
        core_axis_name="c", subcore_axis_name="s",
        num_cores=NUM_CORES, num_subcores=NUM_SUBCORES,
    )

    @functools.partial(
        pl.kernel,
        mesh=mesh,
        out_type=jax.ShapeDtypeStruct((total_rows, D_MODEL), jnp.float32),
        scratch_types=[
            pltpu.VMEM((b_per_w,), jnp.int32),
            pltpu.VMEM((NBUF, CHUNK, D_MODEL), jnp.float32),
            pltpu.SemaphoreType.DMA((NBUF,)),
            pltpu.SemaphoreType.DMA((NBUF,)),
        ],
        compiler_params=pltpu.CompilerParams(use_tc_tiling_on_sc=False),
    )
    def gather(idx_hbm, table_hbm, out_hbm, idx_v, rows_v, g_sem, st_sem):
        wid = lax.axis_index("s") * NUM_CORES + lax.axis_index("c")
        base = wid * b_per_w

        pltpu.sync_copy(idx_hbm.at[pl.ds(base, b_per_w)], idx_v)

        def start_gather(chunk, buf):
            pltpu.async_copy(
                table_hbm.at[idx_v.at[pl.ds(chunk * CHUNK, CHUNK)]],
                rows_v.at[buf], g_sem.at[buf])

        def start_store(chunk, buf):
            pltpu.async_copy(
                rows_v.at[buf], out_hbm.at[pl.ds(base + chunk * CHUNK, CHUNK)],
                st_sem.at[buf])

        def wait(sem, buf):
            pltpu.make_async_copy(
                out_hbm.at[pl.ds(base, CHUNK)], rows_v.at[buf], sem.at[buf]
            ).wait()

        for b in range(INFLIGHT):
            start_gather(b, b)

        @pl.loop(0, n_chunks, step=NBUF)
        def _(o):
            for b in range(NBUF):
                i = o + b
                wait(g_sem, b)
                start_store(i, b)
                j = i + INFLIGHT
                bj = (b + INFLIGHT) % NBUF

                @pl.when(j < n_chunks)
                def _():
                    @pl.when(j >= NBUF)
                    def _():
                        wait(st_sem, bj)
                    start_gather(j, bj)

        for b in range(NBUF):
            wait(st_sem, (n_chunks - NBUF + b) % NBUF)

    return gather


def kernel(x, weight):
    idx = x.reshape(-1).astype(jnp.int32)
    out = _make_gather(idx.shape[0])(idx, weight)
    return out.reshape(x.shape + (D_MODEL,))
